# Initial kernel scaffold; baseline (speedup 1.0000x reference)
import functools

import jax
import jax.numpy as jnp
from jax import lax
from jax.experimental import pallas as pl
from jax.experimental.pallas import tpu as pltpu
from jax.experimental.pallas import tpu_sc as plsc

_N = 10000
_NPAD = 10240
_E = 320000
_NC = 2
_NS = 16
_NW = _NC * _NS
_CH = 128
_NCHUNK = _E // _CH
_CHUNKS_PER_W = _NCHUNK // _NW
_CHUNK_REM = _NCHUNK - _CHUNKS_PER_W * _NW
_RPT = _NPAD // _NS
_BM = 512
_INV_SQRT2 = 0.7071067811865476


def _seg_sum(D, with_deg):
    mesh = plsc.VectorSubcoreMesh(core_axis_name="c", subcore_axis_name="s")
    out_type = [jax.ShapeDtypeStruct((_NC, _NPAD, D), jnp.float32)]
    scratch = [
        pltpu.VMEM((_CH,), jnp.int32),
        pltpu.VMEM((_CH,), jnp.int32),
        pltpu.VMEM((_CH, D), jnp.float32),
        pltpu.VMEM_SHARED((_NPAD, D), jnp.float32),
        pltpu.SemaphoreType.DMA,
    ]
    if with_deg:
        out_type.append(jax.ShapeDtypeStruct((_NC, _NPAD, 1), jnp.float32))
        scratch += [
            pltpu.VMEM((_CH, 1), jnp.float32),
            pltpu.VMEM_SHARED((_NPAD, 1), jnp.float32),
        ]

    def body(*refs):
        if with_deg:
            (p_hbm, src_hbm, dst_hbm, z_hbm, zd_hbm, out_hbm, deg_hbm,
             src_v, dst_v, rows_v, acc, sem, ones_v, dacc) = refs
        else:
            (p_hbm, src_hbm, dst_hbm, z_hbm, out_hbm,
             src_v, dst_v, rows_v, acc, sem) = refs
        c = lax.axis_index("c")
        s = lax.axis_index("s")
        wid = s * _NC + c
        r0 = s * _RPT
        pltpu.sync_copy(z_hbm.at[pl.ds(r0, _RPT)], acc.at[pl.ds(r0, _RPT)])
        if with_deg:
            pltpu.sync_copy(zd_hbm.at[pl.ds(r0, _RPT)], dacc.at[pl.ds(r0, _RPT)])
            for i in range(_CH // 16):
                ones_v[pl.ds(i * 16, 16), 0] = jnp.ones((16,), jnp.float32)
        plsc.subcore_barrier()

        nch = _CHUNKS_PER_W + jnp.where(wid < _CHUNK_REM, 1, 0)

        def step(j, carry):
            base = (wid + j * _NW) * _CH
            pltpu.sync_copy(src_hbm.at[pl.ds(base, _CH)], src_v)
            pltpu.sync_copy(dst_hbm.at[pl.ds(base, _CH)], dst_v)
            pltpu.async_copy(p_hbm.at[src_v], rows_v, sem).wait()
            pltpu.sync_copy(rows_v, acc.at[dst_v], add=True)
            if with_deg:
                pltpu.sync_copy(ones_v, dacc.at[dst_v], add=True)
            return carry

        lax.fori_loop(0, nch, step, 0)
        plsc.subcore_barrier()
        pltpu.sync_copy(acc.at[pl.ds(r0, _RPT)], out_hbm.at[c, pl.ds(r0, _RPT)])
        if with_deg:
            pltpu.sync_copy(dacc.at[pl.ds(r0, _RPT)],
                            deg_hbm.at[c, pl.ds(r0, _RPT)])

    return pl.kernel(body, out_type=out_type, mesh=mesh, scratch_types=scratch)


def _mm_body(x_ref, w_ref, o_ref):
    o_ref[...] = jnp.dot(x_ref[...], w_ref[...],
                         preferred_element_type=jnp.float32)


def _matmul(x, w):
    n, k = x.shape
    m = w.shape[1]
    return pl.pallas_call(
        _mm_body,
        grid=(n // _BM,),
        in_specs=[pl.BlockSpec((_BM, k), lambda i: (i, 0)),
                  pl.BlockSpec((k, m), lambda i: (0, 0))],
        out_specs=pl.BlockSpec((_BM, m), lambda i: (i, 0)),
        out_shape=jax.ShapeDtypeStruct((n, m), jnp.float32),
    )(x, w)


def _layer_body(s_ref, deg_ref, h_ref, wr_ref, b_ref, g_ref, be_ref, wn_ref,
                ho_ref, po_ref):
    degc = jnp.maximum(deg_ref[0] + deg_ref[1], 1.0)
    agg = (s_ref[0] + s_ref[1]) / degc
    pre = agg + b_ref[...] + jnp.dot(h_ref[...], wr_ref[...],
                                     preferred_element_type=jnp.float32)
    mu = jnp.mean(pre, axis=-1, keepdims=True)
    var = jnp.mean((pre - mu) ** 2, axis=-1, keepdims=True)
    h = (pre - mu) * lax.rsqrt(var + 1e-5) * g_ref[...] + be_ref[...]
    h = h * 0.5 * (1.0 + lax.erf(h * _INV_SQRT2))
    ho_ref[...] = h
    po_ref[...] = jnp.dot(h, wn_ref[...], preferred_element_type=jnp.float32)


def _layer(s, deg, h_prev, wr, b, g, be, wn):
    din = h_prev.shape[1]
    d = wr.shape[1]
    dn = wn.shape[1]
    return pl.pallas_call(
        _layer_body,
        grid=(_NPAD // _BM,),
        in_specs=[
            pl.BlockSpec((_NC, _BM, d), lambda i: (0, i, 0)),
            pl.BlockSpec((_NC, _BM, 1), lambda i: (0, i, 0)),
            pl.BlockSpec((_BM, din), lambda i: (i, 0)),
            pl.BlockSpec((din, d), lambda i: (0, 0)),
            pl.BlockSpec((1, d), lambda i: (0, 0)),
            pl.BlockSpec((1, d), lambda i: (0, 0)),
            pl.BlockSpec((1, d), lambda i: (0, 0)),
            pl.BlockSpec((d, dn), lambda i: (0, 0)),
        ],
        out_specs=[pl.BlockSpec((_BM, d), lambda i: (i, 0)),
                   pl.BlockSpec((_BM, dn), lambda i: (i, 0))],
        out_shape=[jax.ShapeDtypeStruct((_NPAD, d), jnp.float32),
                   jax.ShapeDtypeStruct((_NPAD, dn), jnp.float32)],
    )(s, deg, h_prev, wr, b.reshape(1, d), g.reshape(1, d), be.reshape(1, d),
      wn)


def _final_body(s_ref, deg_ref, h_ref, wr_ref, b_ref, o_ref):
    degc = jnp.maximum(deg_ref[0] + deg_ref[1], 1.0)
    agg = (s_ref[0] + s_ref[1]) / degc
    o_ref[...] = agg + b_ref[...] + jnp.dot(h_ref[...], wr_ref[...],
                                            preferred_element_type=jnp.float32)


def _final(s, deg, h_prev, wr, b):
    din = h_prev.shape[1]
    return pl.pallas_call(
        _final_body,
        grid=(_NPAD // _BM,),
        in_specs=[
            pl.BlockSpec((_NC, _BM, 1), lambda i: (0, i, 0)),
            pl.BlockSpec((_NC, _BM, 1), lambda i: (0, i, 0)),
            pl.BlockSpec((_BM, din), lambda i: (i, 0)),
            pl.BlockSpec((din, 1), lambda i: (0, 0)),
            pl.BlockSpec((1, 1), lambda i: (0, 0)),
        ],
        out_specs=pl.BlockSpec((_BM, 1), lambda i: (i, 0)),
        out_shape=jax.ShapeDtypeStruct((_NPAD, 1), jnp.float32),
    )(s, deg, h_prev, wr, b.reshape(1, 1))


def kernel(x, edge_index, W1l, b1, W1r, g1, be1, W2l, b2, W2r, g2, be2,
           W3l, b3, W3r):
    src = edge_index[0]
    dst = edge_index[1]
    xp = jnp.pad(x, ((0, _NPAD - _N), (0, 0)))
    z64 = jnp.zeros((_NPAD, 64), jnp.float32)
    z32 = jnp.zeros((_NPAD, 32), jnp.float32)
    z1 = jnp.zeros((_NPAD, 1), jnp.float32)

    seg1 = _seg_sum(64, with_deg=True)
    seg2 = _seg_sum(32, with_deg=False)
    seg3 = _seg_sum(1, with_deg=False)

    p1 = _matmul(xp, W1l)
    s1, deg = seg1(p1, src, dst, z64, z1)
    h1, p2 = _layer(s1, deg, xp, W1r, b1, g1, be1, W2l)
    s2, = seg2(p2, src, dst, z32)
    h2, p3 = _layer(s2, deg, h1, W2r, b2, g2, be2, W3l)
    s3, = seg3(p3, src, dst, z1)
    out = _final(s3, deg, h2, W3r, b3)
    return out[:_N, 0]


# speedup vs baseline: 6.9694x; 6.9694x over previous
import functools

import jax
import jax.numpy as jnp
from jax import lax
from jax.experimental import pallas as pl
from jax.experimental.pallas import tpu as pltpu
from jax.experimental.pallas import tpu_sc as plsc

_N = 10000
_NPAD = 10240
_E = 320000
_NC = 2
_NS = 16
_NW = _NC * _NS
_CH = 128
_NCHUNK = _E // _CH
_CHUNKS_PER_W = _NCHUNK // _NW
_CHUNK_REM = _NCHUNK - _CHUNKS_PER_W * _NW
_RPT = _NPAD // _NS
_BM = 512
_INV_SQRT2 = 0.7071067811865476


def _seg_sum(D, with_deg):
    mesh = plsc.VectorSubcoreMesh(core_axis_name="c", subcore_axis_name="s",
                                  num_cores=_NC, num_subcores=_NS)
    acc_shape = (_NPAD, D) if D > 1 else (_NPAD,)
    row_shape = (_CH, D) if D > 1 else (_CH,)
    out_type = [jax.ShapeDtypeStruct((_NC,) + acc_shape, jnp.float32)]
    scratch = [
        pltpu.VMEM((_CH,), jnp.int32),
        pltpu.VMEM((_CH,), jnp.int32),
        pltpu.VMEM(row_shape, jnp.float32),
        pltpu.VMEM_SHARED(acc_shape, jnp.float32),
        pltpu.SemaphoreType.DMA,
    ]
    if with_deg:
        out_type.append(jax.ShapeDtypeStruct((_NC, _NPAD), jnp.float32))
        scratch += [
            pltpu.VMEM((_CH,), jnp.float32),
            pltpu.VMEM_SHARED((_NPAD,), jnp.float32),
        ]

    def body(*refs):
        if with_deg:
            (p_hbm, src_hbm, dst_hbm, z_hbm, zd_hbm, out_hbm, deg_hbm,
             src_v, dst_v, rows_v, acc, sem, ones_v, dacc) = refs
        else:
            (p_hbm, src_hbm, dst_hbm, z_hbm, out_hbm,
             src_v, dst_v, rows_v, acc, sem) = refs
        c = lax.axis_index("c")
        s = lax.axis_index("s")
        wid = s * _NC + c
        r0 = s * _RPT
        pltpu.sync_copy(z_hbm.at[pl.ds(r0, _RPT)], acc.at[pl.ds(r0, _RPT)])
        if with_deg:
            pltpu.sync_copy(zd_hbm.at[pl.ds(r0, _RPT)], dacc.at[pl.ds(r0, _RPT)])
            for i in range(_CH // 16):
                ones_v[pl.ds(i * 16, 16)] = jnp.ones((16,), jnp.float32)
        plsc.subcore_barrier()

        nch = _CHUNKS_PER_W + jnp.where(wid < _CHUNK_REM, 1, 0)

        def step(j, carry):
            base = (wid + j * _NW) * _CH
            pltpu.sync_copy(src_hbm.at[pl.ds(base, _CH)], src_v)
            pltpu.sync_copy(dst_hbm.at[pl.ds(base, _CH)], dst_v)
            pltpu.async_copy(p_hbm.at[src_v], rows_v, sem).wait()
            pltpu.sync_copy(rows_v, acc.at[dst_v], add=True)
            if with_deg:
                pltpu.sync_copy(ones_v, dacc.at[dst_v], add=True)
            return carry

        lax.fori_loop(0, nch, step, 0)
        plsc.subcore_barrier()
        pltpu.sync_copy(acc.at[pl.ds(r0, _RPT)], out_hbm.at[c, pl.ds(r0, _RPT)])
        if with_deg:
            pltpu.sync_copy(dacc.at[pl.ds(r0, _RPT)],
                            deg_hbm.at[c, pl.ds(r0, _RPT)])

    return pl.kernel(
        body, out_type=out_type, mesh=mesh, scratch_types=scratch,
        compiler_params=pltpu.CompilerParams(use_tc_tiling_on_sc=False))


def _mm_body(x_ref, w_ref, o_ref):
    o_ref[...] = jnp.dot(x_ref[...], w_ref[...],
                         preferred_element_type=jnp.float32)


def _matmul(x, w):
    n, k = x.shape
    m = w.shape[1]
    return pl.pallas_call(
        _mm_body,
        grid=(n // _BM,),
        in_specs=[pl.BlockSpec((_BM, k), lambda i: (i, 0)),
                  pl.BlockSpec((k, m), lambda i: (0, 0))],
        out_specs=pl.BlockSpec((_BM, m), lambda i: (i, 0)),
        out_shape=jax.ShapeDtypeStruct((n, m), jnp.float32),
    )(x, w)


def _layer_body(s_ref, deg_ref, h_ref, wr_ref, b_ref, g_ref, be_ref, wn_ref,
                ho_ref, po_ref):
    degc = jnp.maximum(deg_ref[0] + deg_ref[1], 1.0)
    agg = (s_ref[0] + s_ref[1]) / degc
    pre = agg + b_ref[...] + jnp.dot(h_ref[...], wr_ref[...],
                                     preferred_element_type=jnp.float32)
    mu = jnp.mean(pre, axis=-1, keepdims=True)
    var = jnp.mean((pre - mu) ** 2, axis=-1, keepdims=True)
    h = (pre - mu) * lax.rsqrt(var + 1e-5) * g_ref[...] + be_ref[...]
    h = h * 0.5 * (1.0 + lax.erf(h * _INV_SQRT2))
    ho_ref[...] = h
    po_ref[...] = jnp.dot(h, wn_ref[...], preferred_element_type=jnp.float32)


def _layer(s, deg, h_prev, wr, b, g, be, wn):
    din = h_prev.shape[1]
    d = wr.shape[1]
    dn = wn.shape[1]
    return pl.pallas_call(
        _layer_body,
        grid=(_NPAD // _BM,),
        in_specs=[
            pl.BlockSpec((_NC, _BM, d), lambda i: (0, i, 0)),
            pl.BlockSpec((_NC, _BM, 1), lambda i: (0, i, 0)),
            pl.BlockSpec((_BM, din), lambda i: (i, 0)),
            pl.BlockSpec((din, d), lambda i: (0, 0)),
            pl.BlockSpec((1, d), lambda i: (0, 0)),
            pl.BlockSpec((1, d), lambda i: (0, 0)),
            pl.BlockSpec((1, d), lambda i: (0, 0)),
            pl.BlockSpec((d, dn), lambda i: (0, 0)),
        ],
        out_specs=[pl.BlockSpec((_BM, d), lambda i: (i, 0)),
                   pl.BlockSpec((_BM, dn), lambda i: (i, 0))],
        out_shape=[jax.ShapeDtypeStruct((_NPAD, d), jnp.float32),
                   jax.ShapeDtypeStruct((_NPAD, dn), jnp.float32)],
    )(s, deg, h_prev, wr, b.reshape(1, d), g.reshape(1, d), be.reshape(1, d),
      wn)


def _final_body(s_ref, deg_ref, h_ref, wr_ref, b_ref, o_ref):
    degc = jnp.maximum(deg_ref[0] + deg_ref[1], 1.0)
    agg = (s_ref[0] + s_ref[1]) / degc
    o_ref[...] = agg + b_ref[...] + jnp.dot(h_ref[...], wr_ref[...],
                                            preferred_element_type=jnp.float32)


def _final(s, deg, h_prev, wr, b):
    din = h_prev.shape[1]
    return pl.pallas_call(
        _final_body,
        grid=(_NPAD // _BM,),
        in_specs=[
            pl.BlockSpec((_NC, _BM, 1), lambda i: (0, i, 0)),
            pl.BlockSpec((_NC, _BM, 1), lambda i: (0, i, 0)),
            pl.BlockSpec((_BM, din), lambda i: (i, 0)),
            pl.BlockSpec((din, 1), lambda i: (0, 0)),
            pl.BlockSpec((1, 1), lambda i: (0, 0)),
        ],
        out_specs=pl.BlockSpec((_BM, 1), lambda i: (i, 0)),
        out_shape=jax.ShapeDtypeStruct((_NPAD, 1), jnp.float32),
    )(s, deg, h_prev, wr, b.reshape(1, 1))


def kernel(x, edge_index, W1l, b1, W1r, g1, be1, W2l, b2, W2r, g2, be2,
           W3l, b3, W3r):
    src = edge_index[0]
    dst = edge_index[1]
    xp = jnp.pad(x, ((0, _NPAD - _N), (0, 0)))
    z64 = jnp.zeros((_NPAD, 64), jnp.float32)
    z32 = jnp.zeros((_NPAD, 32), jnp.float32)
    zd = jnp.zeros((_NPAD,), jnp.float32)

    seg1 = _seg_sum(64, with_deg=True)
    seg2 = _seg_sum(32, with_deg=False)
    seg3 = _seg_sum(1, with_deg=False)

    p1 = _matmul(xp, W1l)
    s1, deg = seg1(p1, src, dst, z64, zd)
    deg = deg.reshape(_NC, _NPAD, 1)
    h1, p2 = _layer(s1, deg, xp, W1r, b1, g1, be1, W2l)
    s2, = seg2(p2, src, dst, z32)
    h2, p3 = _layer(s2, deg, h1, W2r, b2, g2, be2, W3l)
    s3, = seg3(p3.reshape(_NPAD), src, dst, zd)
    out = _final(s3.reshape(_NC, _NPAD, 1), deg, h2, W3r, b3)
    return out[:_N, 0]


# speedup vs baseline: 13.5244x; 1.9405x over previous
import functools

import jax
import jax.numpy as jnp
from jax import lax
from jax.experimental import pallas as pl
from jax.experimental.pallas import tpu as pltpu
from jax.experimental.pallas import tpu_sc as plsc

_N = 10000
_NPAD = 10240
_E = 320000
_NC = 2
_NS = 16
_NW = _NC * _NS
_CH = 128
_NCHUNK = _E // _CH
_CHUNKS_PER_W = _NCHUNK // _NW
_CHUNK_REM = _NCHUNK - _CHUNKS_PER_W * _NW
_RPT = _NPAD // _NS
_BM = 512
_INV_SQRT2 = 0.7071067811865476


def _seg_sum(D, with_deg):
    mesh = plsc.VectorSubcoreMesh(core_axis_name="c", subcore_axis_name="s",
                                  num_cores=_NC, num_subcores=_NS)
    acc_shape = (_NPAD, D) if D > 1 else (_NPAD,)
    row_shape = (_CH, D) if D > 1 else (_CH,)
    out_type = [jax.ShapeDtypeStruct((_NC,) + acc_shape, jnp.float32)]
    scratch = [
        pltpu.VMEM((_CHUNKS_PER_W + 1, _CH), jnp.int32),
        pltpu.VMEM((_CHUNKS_PER_W + 1, _CH), jnp.int32),
        pltpu.VMEM(row_shape, jnp.float32),
        pltpu.VMEM(row_shape, jnp.float32),
        pltpu.VMEM_SHARED(acc_shape, jnp.float32),
        pltpu.SemaphoreType.DMA,
        pltpu.SemaphoreType.DMA,
    ]
    if with_deg:
        out_type.append(jax.ShapeDtypeStruct((_NC, _NPAD), jnp.float32))
        scratch += [
            pltpu.VMEM((_CH,), jnp.float32),
            pltpu.VMEM_SHARED((_NPAD,), jnp.float32),
        ]

    def body(*refs):
        if with_deg:
            (p_hbm, src_hbm, dst_hbm, z_hbm, zd_hbm, out_hbm, deg_hbm,
             src_blk, dst_blk, rows0, rows1, acc, semA, semB,
             ones_v, dacc) = refs
        else:
            (p_hbm, src_hbm, dst_hbm, z_hbm, out_hbm,
             src_blk, dst_blk, rows0, rows1, acc, semA, semB) = refs
        c = lax.axis_index("c")
        s = lax.axis_index("s")
        wid = s * _NC + c
        r0 = s * _RPT
        pltpu.sync_copy(z_hbm.at[pl.ds(r0, _RPT)], acc.at[pl.ds(r0, _RPT)])
        if with_deg:
            pltpu.sync_copy(zd_hbm.at[pl.ds(r0, _RPT)], dacc.at[pl.ds(r0, _RPT)])
            for i in range(_CH // 16):
                ones_v[pl.ds(i * 16, 16)] = jnp.ones((16,), jnp.float32)

        nrows = _CHUNKS_PER_W + jnp.where(wid < _CHUNK_REM, 1, 0)
        row0 = _CHUNKS_PER_W * wid + jnp.minimum(wid, _CHUNK_REM)
        row0c = jnp.minimum(row0, _NCHUNK - (_CHUNKS_PER_W + 1))
        off = row0 - row0c
        pltpu.sync_copy(src_hbm.at[pl.ds(row0c, _CHUNKS_PER_W + 1)], src_blk)
        pltpu.sync_copy(dst_hbm.at[pl.ds(row0c, _CHUNKS_PER_W + 1)], dst_blk)
        plsc.subcore_barrier()

        def scat(bj, rows_v):
            pltpu.sync_copy(rows_v, acc.at[dst_blk.at[bj]], add=True)
            if with_deg:
                pltpu.sync_copy(ones_v, dacc.at[dst_blk.at[bj]], add=True)

        pltpu.async_copy(p_hbm.at[src_blk.at[off]], rows0, semA)

        def step2(k, carry):
            j0 = 2 * k
            j1 = j0 + 1
            pltpu.make_async_copy(p_hbm.at[src_blk.at[off + j0]], rows0,
                                  semA).wait()

            @pl.when(j1 < nrows)
            def _():
                pltpu.async_copy(p_hbm.at[src_blk.at[off + j1]], rows1, semB)

            scat(off + j0, rows0)

            @pl.when(j1 + 1 < nrows)
            def _():
                pltpu.async_copy(p_hbm.at[src_blk.at[off + j1 + 1]], rows0,
                                 semA)

            @pl.when(j1 < nrows)
            def _():
                pltpu.make_async_copy(p_hbm.at[src_blk.at[off + j1]], rows1,
                                      semB).wait()
                scat(off + j1, rows1)

            return carry

        lax.fori_loop(0, (nrows + 1) // 2, step2, 0)
        plsc.subcore_barrier()
        pltpu.sync_copy(acc.at[pl.ds(r0, _RPT)], out_hbm.at[c, pl.ds(r0, _RPT)])
        if with_deg:
            pltpu.sync_copy(dacc.at[pl.ds(r0, _RPT)],
                            deg_hbm.at[c, pl.ds(r0, _RPT)])

    return pl.kernel(
        body, out_type=out_type, mesh=mesh, scratch_types=scratch,
        compiler_params=pltpu.CompilerParams(use_tc_tiling_on_sc=False))


def _mm_body(x_ref, w_ref, o_ref):
    o_ref[...] = jnp.dot(x_ref[...], w_ref[...],
                         preferred_element_type=jnp.float32)


def _matmul(x, w):
    n, k = x.shape
    m = w.shape[1]
    return pl.pallas_call(
        _mm_body,
        grid=(n // _BM,),
        in_specs=[pl.BlockSpec((_BM, k), lambda i: (i, 0)),
                  pl.BlockSpec((k, m), lambda i: (0, 0))],
        out_specs=pl.BlockSpec((_BM, m), lambda i: (i, 0)),
        out_shape=jax.ShapeDtypeStruct((n, m), jnp.float32),
    )(x, w)


def _layer_body(s_ref, deg_ref, h_ref, wr_ref, b_ref, g_ref, be_ref, wn_ref,
                ho_ref, po_ref):
    degc = jnp.maximum(deg_ref[0] + deg_ref[1], 1.0)
    agg = (s_ref[0] + s_ref[1]) / degc
    pre = agg + b_ref[...] + jnp.dot(h_ref[...], wr_ref[...],
                                     preferred_element_type=jnp.float32)
    mu = jnp.mean(pre, axis=-1, keepdims=True)
    var = jnp.mean((pre - mu) ** 2, axis=-1, keepdims=True)
    h = (pre - mu) * lax.rsqrt(var + 1e-5) * g_ref[...] + be_ref[...]
    h = h * 0.5 * (1.0 + lax.erf(h * _INV_SQRT2))
    ho_ref[...] = h
    po_ref[...] = jnp.dot(h, wn_ref[...], preferred_element_type=jnp.float32)


def _layer(s, deg, h_prev, wr, b, g, be, wn):
    din = h_prev.shape[1]
    d = wr.shape[1]
    dn = wn.shape[1]
    return pl.pallas_call(
        _layer_body,
        grid=(_NPAD // _BM,),
        in_specs=[
            pl.BlockSpec((_NC, _BM, d), lambda i: (0, i, 0)),
            pl.BlockSpec((_NC, _BM, 1), lambda i: (0, i, 0)),
            pl.BlockSpec((_BM, din), lambda i: (i, 0)),
            pl.BlockSpec((din, d), lambda i: (0, 0)),
            pl.BlockSpec((1, d), lambda i: (0, 0)),
            pl.BlockSpec((1, d), lambda i: (0, 0)),
            pl.BlockSpec((1, d), lambda i: (0, 0)),
            pl.BlockSpec((d, dn), lambda i: (0, 0)),
        ],
        out_specs=[pl.BlockSpec((_BM, d), lambda i: (i, 0)),
                   pl.BlockSpec((_BM, dn), lambda i: (i, 0))],
        out_shape=[jax.ShapeDtypeStruct((_NPAD, d), jnp.float32),
                   jax.ShapeDtypeStruct((_NPAD, dn), jnp.float32)],
    )(s, deg, h_prev, wr, b.reshape(1, d), g.reshape(1, d), be.reshape(1, d),
      wn)


def _final_body(s_ref, deg_ref, h_ref, wr_ref, b_ref, o_ref):
    degc = jnp.maximum(deg_ref[0] + deg_ref[1], 1.0)
    agg = (s_ref[0] + s_ref[1]) / degc
    o_ref[...] = agg + b_ref[...] + jnp.dot(h_ref[...], wr_ref[...],
                                            preferred_element_type=jnp.float32)


def _final(s, deg, h_prev, wr, b):
    din = h_prev.shape[1]
    return pl.pallas_call(
        _final_body,
        grid=(_NPAD // _BM,),
        in_specs=[
            pl.BlockSpec((_NC, _BM, 1), lambda i: (0, i, 0)),
            pl.BlockSpec((_NC, _BM, 1), lambda i: (0, i, 0)),
            pl.BlockSpec((_BM, din), lambda i: (i, 0)),
            pl.BlockSpec((din, 1), lambda i: (0, 0)),
            pl.BlockSpec((1, 1), lambda i: (0, 0)),
        ],
        out_specs=pl.BlockSpec((_BM, 1), lambda i: (i, 0)),
        out_shape=jax.ShapeDtypeStruct((_NPAD, 1), jnp.float32),
    )(s, deg, h_prev, wr, b.reshape(1, 1))


def kernel(x, edge_index, W1l, b1, W1r, g1, be1, W2l, b2, W2r, g2, be2,
           W3l, b3, W3r):
    src = edge_index[0].reshape(_NCHUNK, _CH)
    dst = edge_index[1].reshape(_NCHUNK, _CH)
    xp = jnp.pad(x, ((0, _NPAD - _N), (0, 0)))
    z64 = jnp.zeros((_NPAD, 64), jnp.float32)
    z32 = jnp.zeros((_NPAD, 32), jnp.float32)
    zd = jnp.zeros((_NPAD,), jnp.float32)

    seg1 = _seg_sum(64, with_deg=True)
    seg2 = _seg_sum(32, with_deg=False)
    seg3 = _seg_sum(1, with_deg=False)

    p1 = _matmul(xp, W1l)
    s1, deg = seg1(p1, src, dst, z64, zd)
    deg = deg.reshape(_NC, _NPAD, 1)
    h1, p2 = _layer(s1, deg, xp, W1r, b1, g1, be1, W2l)
    s2, = seg2(p2, src, dst, z32)
    h2, p3 = _layer(s2, deg, h1, W2r, b2, g2, be2, W3l)
    s3, = seg3(p3.reshape(_NPAD), src, dst, zd)
    out = _final(s3.reshape(_NC, _NPAD, 1), deg, h2, W3r, b3)
    return out[:_N, 0]


# speedup vs baseline: 14.4508x; 1.0685x over previous
import functools

import jax
import jax.numpy as jnp
from jax import lax
from jax.experimental import pallas as pl
from jax.experimental.pallas import tpu as pltpu
from jax.experimental.pallas import tpu_sc as plsc

_N = 10000
_NPAD = 10240
_E = 320000
_NC = 2
_NS = 16
_NW = _NC * _NS
_CH = 128
_NCHUNK = _E // _CH
_CHUNKS_PER_W = _NCHUNK // _NW
_CHUNK_REM = _NCHUNK - _CHUNKS_PER_W * _NW
_RPT = _NPAD // _NS
_BM = 512
_INV_SQRT2 = 0.7071067811865476


_G = 2
_NGRP = _CHUNKS_PER_W // _G
_GCH = _G * _CH


def _seg_sum(D, with_deg):
    mesh = plsc.VectorSubcoreMesh(core_axis_name="c", subcore_axis_name="s",
                                  num_cores=_NC, num_subcores=_NS)
    acc_shape = (_NPAD, D) if D > 1 else (_NPAD,)
    buf_shape = (_GCH, D) if D > 1 else (_GCH,)
    out_type = [jax.ShapeDtypeStruct((_NC,) + acc_shape, jnp.float32)]
    scratch = [
        pltpu.VMEM((_CHUNKS_PER_W + 1, _CH), jnp.int32),
        pltpu.VMEM((_CHUNKS_PER_W + 1, _CH), jnp.int32),
        [pltpu.VMEM(buf_shape, jnp.float32)] * 4,
        pltpu.VMEM_SHARED(acc_shape, jnp.float32),
        [pltpu.SemaphoreType.DMA] * 4,
        [pltpu.SemaphoreType.DMA] * 4,
    ]
    if with_deg:
        out_type.append(jax.ShapeDtypeStruct((_NC, _NPAD), jnp.float32))
        scratch += [
            pltpu.VMEM((_CH,), jnp.float32),
            pltpu.VMEM_SHARED((_NPAD,), jnp.float32),
        ]

    def body(*refs):
        if with_deg:
            (p_hbm, src_hbm, dst_hbm, z_hbm, zd_hbm, out_hbm, deg_hbm,
             src_blk, dst_blk, bufs, acc, gsem, ssem, ones_v, dacc) = refs
        else:
            (p_hbm, src_hbm, dst_hbm, z_hbm, out_hbm,
             src_blk, dst_blk, bufs, acc, gsem, ssem) = refs
        c = lax.axis_index("c")
        s = lax.axis_index("s")
        wid = s * _NC + c
        r0 = s * _RPT
        pltpu.sync_copy(z_hbm.at[pl.ds(r0, _RPT)], acc.at[pl.ds(r0, _RPT)])
        if with_deg:
            pltpu.sync_copy(zd_hbm.at[pl.ds(r0, _RPT)], dacc.at[pl.ds(r0, _RPT)])
            for i in range(_CH // 16):
                ones_v[pl.ds(i * 16, 16)] = jnp.ones((16,), jnp.float32)

        row0 = _CHUNKS_PER_W * wid
        pltpu.sync_copy(src_hbm.at[pl.ds(row0, _CHUNKS_PER_W)],
                        src_blk.at[pl.ds(0, _CHUNKS_PER_W)])
        pltpu.sync_copy(dst_hbm.at[pl.ds(row0, _CHUNKS_PER_W)],
                        dst_blk.at[pl.ds(0, _CHUNKS_PER_W)])

        @pl.when(wid < _CHUNK_REM)
        def _():
            tr = _CHUNKS_PER_W * _NW + wid
            pltpu.sync_copy(src_hbm.at[pl.ds(tr, 1)],
                            src_blk.at[pl.ds(_CHUNKS_PER_W, 1)])
            pltpu.sync_copy(dst_hbm.at[pl.ds(tr, 1)],
                            dst_blk.at[pl.ds(_CHUNKS_PER_W, 1)])

        plsc.subcore_barrier()

        def fire_gathers(g, i):
            for t in range(_G):
                pltpu.async_copy(p_hbm.at[src_blk.at[g * _G + t]],
                                 bufs[i].at[pl.ds(t * _CH, _CH)], gsem[i])

        def drain_gathers(g, i):
            for t in range(_G):
                pltpu.make_async_copy(p_hbm.at[src_blk.at[g * _G + t]],
                                      bufs[i].at[pl.ds(t * _CH, _CH)],
                                      gsem[i]).wait()

        def fire_scatters(g, i):
            for t in range(_G):
                pltpu.async_copy(bufs[i].at[pl.ds(t * _CH, _CH)],
                                 acc.at[dst_blk.at[g * _G + t]], ssem[i],
                                 add=True)
                if with_deg:
                    pltpu.async_copy(ones_v, dacc.at[dst_blk.at[g * _G + t]],
                                     ssem[i], add=True)

        def drain_scatters(g, i):
            for t in range(_G):
                pltpu.make_async_copy(bufs[i].at[pl.ds(t * _CH, _CH)],
                                      acc.at[dst_blk.at[g * _G + t]],
                                      ssem[i]).wait()
                if with_deg:
                    pltpu.make_async_copy(ones_v,
                                          dacc.at[dst_blk.at[g * _G + t]],
                                          ssem[i]).wait()

        fire_gathers(0, 0)
        fire_gathers(1, 1)
        for g in range(4):
            i = g
            i2 = (i + 2) % 4
            drain_gathers(g, i)
            fire_scatters(g, i)
            if g >= 2:
                drain_scatters(g - 2, i2)
            fire_gathers(g + 2, i2)

        def step(k, carry):
            for i in range(4):
                g = 4 * k + i
                i2 = (i + 2) % 4
                drain_gathers(g, i)
                fire_scatters(g, i)
                drain_scatters(g - 2, i2)
                fire_gathers(g + 2, i2)
            return carry

        lax.fori_loop(1, _NGRP // 4, step, 0)

        for g in range(4 * (_NGRP // 4), _NGRP):
            i = g % 4
            i2 = (i + 2) % 4
            drain_gathers(g, i)
            fire_scatters(g, i)
            if g + 2 < _NGRP:
                drain_scatters(g - 2, i2)
                fire_gathers(g + 2, i2)
        for g in range(_NGRP - 4, _NGRP):
            drain_scatters(g, g % 4)

        @pl.when(wid < _CHUNK_REM)
        def _():
            pltpu.async_copy(p_hbm.at[src_blk.at[_CHUNKS_PER_W]],
                             bufs[0].at[pl.ds(0, _CH)], gsem[0])
            pltpu.make_async_copy(p_hbm.at[src_blk.at[_CHUNKS_PER_W]],
                                  bufs[0].at[pl.ds(0, _CH)], gsem[0]).wait()
            pltpu.sync_copy(bufs[0].at[pl.ds(0, _CH)],
                            acc.at[dst_blk.at[_CHUNKS_PER_W]], add=True)
            if with_deg:
                pltpu.sync_copy(ones_v, dacc.at[dst_blk.at[_CHUNKS_PER_W]],
                                add=True)

        plsc.subcore_barrier()
        pltpu.sync_copy(acc.at[pl.ds(r0, _RPT)], out_hbm.at[c, pl.ds(r0, _RPT)])
        if with_deg:
            pltpu.sync_copy(dacc.at[pl.ds(r0, _RPT)],
                            deg_hbm.at[c, pl.ds(r0, _RPT)])

    return pl.kernel(
        body, out_type=out_type, mesh=mesh, scratch_types=scratch,
        compiler_params=pltpu.CompilerParams(use_tc_tiling_on_sc=False))


def _mm_body(x_ref, w_ref, o_ref):
    o_ref[...] = jnp.dot(x_ref[...], w_ref[...],
                         preferred_element_type=jnp.float32)


def _matmul(x, w):
    n, k = x.shape
    m = w.shape[1]
    return pl.pallas_call(
        _mm_body,
        grid=(n // _BM,),
        in_specs=[pl.BlockSpec((_BM, k), lambda i: (i, 0)),
                  pl.BlockSpec((k, m), lambda i: (0, 0))],
        out_specs=pl.BlockSpec((_BM, m), lambda i: (i, 0)),
        out_shape=jax.ShapeDtypeStruct((n, m), jnp.float32),
    )(x, w)


def _layer_body(s_ref, deg_ref, h_ref, wr_ref, b_ref, g_ref, be_ref, wn_ref,
                ho_ref, po_ref):
    degc = jnp.maximum(deg_ref[0] + deg_ref[1], 1.0)
    agg = (s_ref[0] + s_ref[1]) / degc
    pre = agg + b_ref[...] + jnp.dot(h_ref[...], wr_ref[...],
                                     preferred_element_type=jnp.float32)
    mu = jnp.mean(pre, axis=-1, keepdims=True)
    var = jnp.mean((pre - mu) ** 2, axis=-1, keepdims=True)
    h = (pre - mu) * lax.rsqrt(var + 1e-5) * g_ref[...] + be_ref[...]
    h = h * 0.5 * (1.0 + lax.erf(h * _INV_SQRT2))
    ho_ref[...] = h
    po_ref[...] = jnp.dot(h, wn_ref[...], preferred_element_type=jnp.float32)


def _layer(s, deg, h_prev, wr, b, g, be, wn):
    din = h_prev.shape[1]
    d = wr.shape[1]
    dn = wn.shape[1]
    return pl.pallas_call(
        _layer_body,
        grid=(_NPAD // _BM,),
        in_specs=[
            pl.BlockSpec((_NC, _BM, d), lambda i: (0, i, 0)),
            pl.BlockSpec((_NC, _BM, 1), lambda i: (0, i, 0)),
            pl.BlockSpec((_BM, din), lambda i: (i, 0)),
            pl.BlockSpec((din, d), lambda i: (0, 0)),
            pl.BlockSpec((1, d), lambda i: (0, 0)),
            pl.BlockSpec((1, d), lambda i: (0, 0)),
            pl.BlockSpec((1, d), lambda i: (0, 0)),
            pl.BlockSpec((d, dn), lambda i: (0, 0)),
        ],
        out_specs=[pl.BlockSpec((_BM, d), lambda i: (i, 0)),
                   pl.BlockSpec((_BM, dn), lambda i: (i, 0))],
        out_shape=[jax.ShapeDtypeStruct((_NPAD, d), jnp.float32),
                   jax.ShapeDtypeStruct((_NPAD, dn), jnp.float32)],
    )(s, deg, h_prev, wr, b.reshape(1, d), g.reshape(1, d), be.reshape(1, d),
      wn)


def _final_body(s_ref, deg_ref, h_ref, wr_ref, b_ref, o_ref):
    degc = jnp.maximum(deg_ref[0] + deg_ref[1], 1.0)
    agg = (s_ref[0] + s_ref[1]) / degc
    o_ref[...] = agg + b_ref[...] + jnp.dot(h_ref[...], wr_ref[...],
                                            preferred_element_type=jnp.float32)


def _final(s, deg, h_prev, wr, b):
    din = h_prev.shape[1]
    return pl.pallas_call(
        _final_body,
        grid=(_NPAD // _BM,),
        in_specs=[
            pl.BlockSpec((_NC, _BM, 1), lambda i: (0, i, 0)),
            pl.BlockSpec((_NC, _BM, 1), lambda i: (0, i, 0)),
            pl.BlockSpec((_BM, din), lambda i: (i, 0)),
            pl.BlockSpec((din, 1), lambda i: (0, 0)),
            pl.BlockSpec((1, 1), lambda i: (0, 0)),
        ],
        out_specs=pl.BlockSpec((_BM, 1), lambda i: (i, 0)),
        out_shape=jax.ShapeDtypeStruct((_NPAD, 1), jnp.float32),
    )(s, deg, h_prev, wr, b.reshape(1, 1))


def kernel(x, edge_index, W1l, b1, W1r, g1, be1, W2l, b2, W2r, g2, be2,
           W3l, b3, W3r):
    src = edge_index[0].reshape(_NCHUNK, _CH)
    dst = edge_index[1].reshape(_NCHUNK, _CH)
    xp = jnp.pad(x, ((0, _NPAD - _N), (0, 0)))
    z64 = jnp.zeros((_NPAD, 64), jnp.float32)
    z32 = jnp.zeros((_NPAD, 32), jnp.float32)
    zd = jnp.zeros((_NPAD,), jnp.float32)

    seg1 = _seg_sum(64, with_deg=True)
    seg2 = _seg_sum(32, with_deg=False)
    seg3 = _seg_sum(1, with_deg=False)

    p1 = _matmul(xp, W1l)
    s1, deg = seg1(p1, src, dst, z64, zd)
    deg = deg.reshape(_NC, _NPAD, 1)
    h1, p2 = _layer(s1, deg, xp, W1r, b1, g1, be1, W2l)
    s2, = seg2(p2, src, dst, z32)
    h2, p3 = _layer(s2, deg, h1, W2r, b2, g2, be2, W3l)
    s3, = seg3(p3.reshape(_NPAD), src, dst, zd)
    out = _final(s3.reshape(_NC, _NPAD, 1), deg, h2, W3r, b3)
    return out[:_N, 0]


# speedup vs baseline: 17.5751x; 1.2162x over previous
import functools

import jax
import jax.numpy as jnp
from jax import lax
from jax.experimental import pallas as pl
from jax.experimental.pallas import tpu as pltpu
from jax.experimental.pallas import tpu_sc as plsc

_N = 10000
_NPAD = 10240
_E = 320000
_NC = 2
_NS = 16
_NW = _NC * _NS
_CH = 128
_NCHUNK = _E // _CH
_CPW = 78
_CHUNK_REM = _NCHUNK - _CPW * _NW
_EPW = _CPW * _CH
_RPT = _NPAD // _NS
_BM = 512
_INV_SQRT2 = 0.7071067811865476

_G = 2
_NGRP = _CPW // _G
_GCH = _G * _CH


def _seg_sum(D):
    mesh = plsc.VectorSubcoreMesh(core_axis_name="c", subcore_axis_name="s",
                                  num_cores=_NC, num_subcores=_NS)
    out_type = [jax.ShapeDtypeStruct((_NC, _NPAD, D), jnp.float32)]
    scratch = [
        pltpu.VMEM((_CPW + 1, _CH), jnp.int32),
        pltpu.VMEM((_CPW + 1, _CH), jnp.int32),
        [pltpu.VMEM((_GCH, D), jnp.float32)] * 4,
        pltpu.VMEM_SHARED((_NPAD, D), jnp.float32),
        [pltpu.SemaphoreType.DMA] * 4,
        [pltpu.SemaphoreType.DMA] * 4,
    ]

    def body(p_hbm, src_hbm, dst_hbm, z_hbm, out_hbm,
             src_blk, dst_blk, bufs, acc, gsem, ssem):
        c = lax.axis_index("c")
        s = lax.axis_index("s")
        wid = s * _NC + c
        r0 = s * _RPT
        pltpu.sync_copy(z_hbm.at[pl.ds(r0, _RPT)], acc.at[pl.ds(r0, _RPT)])

        row0 = _CPW * wid
        pltpu.sync_copy(src_hbm.at[pl.ds(row0, _CPW)],
                        src_blk.at[pl.ds(0, _CPW)])
        pltpu.sync_copy(dst_hbm.at[pl.ds(row0, _CPW)],
                        dst_blk.at[pl.ds(0, _CPW)])

        @pl.when(wid < _CHUNK_REM)
        def _():
            tr = _CPW * _NW + wid
            pltpu.sync_copy(src_hbm.at[pl.ds(tr, 1)],
                            src_blk.at[pl.ds(_CPW, 1)])
            pltpu.sync_copy(dst_hbm.at[pl.ds(tr, 1)],
                            dst_blk.at[pl.ds(_CPW, 1)])

        plsc.subcore_barrier()

        def fire_gathers(g, i):
            for t in range(_G):
                pltpu.async_copy(p_hbm.at[src_blk.at[g * _G + t]],
                                 bufs[i].at[pl.ds(t * _CH, _CH)], gsem[i])

        def drain_gathers(g, i):
            for t in range(_G):
                pltpu.make_async_copy(p_hbm.at[src_blk.at[g * _G + t]],
                                      bufs[i].at[pl.ds(t * _CH, _CH)],
                                      gsem[i]).wait()

        def fire_scatters(g, i):
            for t in range(_G):
                pltpu.async_copy(bufs[i].at[pl.ds(t * _CH, _CH)],
                                 acc.at[dst_blk.at[g * _G + t]], ssem[i],
                                 add=True)

        def drain_scatters(g, i):
            for t in range(_G):
                pltpu.make_async_copy(bufs[i].at[pl.ds(t * _CH, _CH)],
                                      acc.at[dst_blk.at[g * _G + t]],
                                      ssem[i]).wait()

        fire_gathers(0, 0)
        fire_gathers(1, 1)
        for g in range(4):
            i = g
            i2 = (i + 2) % 4
            drain_gathers(g, i)
            fire_scatters(g, i)
            if g >= 2:
                drain_scatters(g - 2, i2)
            fire_gathers(g + 2, i2)

        def step(k, carry):
            for i in range(4):
                g = 4 * k + i
                i2 = (i + 2) % 4
                drain_gathers(g, i)
                fire_scatters(g, i)
                drain_scatters(g - 2, i2)
                fire_gathers(g + 2, i2)
            return carry

        lax.fori_loop(1, _NGRP // 4, step, 0)

        for g in range(4 * (_NGRP // 4), _NGRP):
            i = g % 4
            i2 = (i + 2) % 4
            drain_gathers(g, i)
            fire_scatters(g, i)
            if g + 2 < _NGRP:
                drain_scatters(g - 2, i2)
                fire_gathers(g + 2, i2)
        for g in range(_NGRP - 4, _NGRP):
            drain_scatters(g, g % 4)

        @pl.when(wid < _CHUNK_REM)
        def _():
            pltpu.async_copy(p_hbm.at[src_blk.at[_CPW]],
                             bufs[0].at[pl.ds(0, _CH)], gsem[0])
            pltpu.make_async_copy(p_hbm.at[src_blk.at[_CPW]],
                                  bufs[0].at[pl.ds(0, _CH)], gsem[0]).wait()
            pltpu.sync_copy(bufs[0].at[pl.ds(0, _CH)],
                            acc.at[dst_blk.at[_CPW]], add=True)

        plsc.subcore_barrier()
        pltpu.sync_copy(acc.at[pl.ds(r0, _RPT)], out_hbm.at[c, pl.ds(r0, _RPT)])

    return pl.kernel(
        body, out_type=out_type, mesh=mesh, scratch_types=scratch,
        compiler_params=pltpu.CompilerParams(use_tc_tiling_on_sc=False))


def _seg_deg():
    mesh = plsc.VectorSubcoreMesh(core_axis_name="c", subcore_axis_name="s",
                                  num_cores=_NC, num_subcores=_NS)
    out_type = [jax.ShapeDtypeStruct((_NW, _NPAD), jnp.float32)]
    scratch = [
        pltpu.VMEM((_NPAD,), jnp.float32),
        pltpu.VMEM((_EPW,), jnp.int32),
        pltpu.VMEM((_CH,), jnp.int32),
    ]

    def body(dst_hbm, z_hbm, out_hbm, accl, dstm, dstt):
        c = lax.axis_index("c")
        s = lax.axis_index("s")
        wid = s * _NC + c
        pltpu.sync_copy(z_hbm, accl)
        e0 = wid * _EPW
        pltpu.sync_copy(dst_hbm.at[pl.ds(e0, _EPW)], dstm)

        @pl.when(wid < _CHUNK_REM)
        def _():
            t0 = _EPW * _NW + _CH * wid
            pltpu.sync_copy(dst_hbm.at[pl.ds(t0, _CH)], dstt)

        ones16 = jnp.ones((16,), jnp.float32)

        def step(i, carry):
            d16 = dstm[pl.ds(i * 16, 16)]
            plsc.addupdate_scatter(accl, [d16], ones16)
            return carry

        lax.fori_loop(0, _EPW // 16, step, 0)

        @pl.when(wid < _CHUNK_REM)
        def _():
            def stept(i, carry):
                d16 = dstt[pl.ds(i * 16, 16)]
                plsc.addupdate_scatter(accl, [d16], ones16)
                return carry

            lax.fori_loop(0, _CH // 16, stept, 0)

        pltpu.sync_copy(accl, out_hbm.at[wid])

    return pl.kernel(
        body, out_type=out_type, mesh=mesh, scratch_types=scratch,
        compiler_params=pltpu.CompilerParams(use_tc_tiling_on_sc=False,
                                             needs_layout_passes=False))


def _seg_scalar():
    mesh = plsc.VectorSubcoreMesh(core_axis_name="c", subcore_axis_name="s",
                                  num_cores=_NC, num_subcores=_NS)
    out_type = [jax.ShapeDtypeStruct((_NW, _NPAD), jnp.float32)]
    scratch = [
        pltpu.VMEM((_NPAD,), jnp.float32),
        pltpu.VMEM((_NPAD,), jnp.float32),
        pltpu.VMEM((_EPW,), jnp.int32),
        pltpu.VMEM((_EPW,), jnp.int32),
        pltpu.VMEM((_CH,), jnp.int32),
        pltpu.VMEM((_CH,), jnp.int32),
    ]

    def body(p_hbm, src_hbm, dst_hbm, z_hbm, out_hbm,
             p3_v, accl, srcm, dstm, srct, dstt):
        c = lax.axis_index("c")
        s = lax.axis_index("s")
        wid = s * _NC + c
        pltpu.sync_copy(p_hbm, p3_v)
        pltpu.sync_copy(z_hbm, accl)
        e0 = wid * _EPW
        pltpu.sync_copy(src_hbm.at[pl.ds(e0, _EPW)], srcm)
        pltpu.sync_copy(dst_hbm.at[pl.ds(e0, _EPW)], dstm)

        @pl.when(wid < _CHUNK_REM)
        def _():
            t0 = _EPW * _NW + _CH * wid
            pltpu.sync_copy(src_hbm.at[pl.ds(t0, _CH)], srct)
            pltpu.sync_copy(dst_hbm.at[pl.ds(t0, _CH)], dstt)

        def step(i, carry):
            s16 = srcm[pl.ds(i * 16, 16)]
            d16 = dstm[pl.ds(i * 16, 16)]
            v = plsc.load_gather(p3_v, [s16])
            plsc.addupdate_scatter(accl, [d16], v)
            return carry

        lax.fori_loop(0, _EPW // 16, step, 0)

        @pl.when(wid < _CHUNK_REM)
        def _():
            def stept(i, carry):
                s16 = srct[pl.ds(i * 16, 16)]
                d16 = dstt[pl.ds(i * 16, 16)]
                v = plsc.load_gather(p3_v, [s16])
                plsc.addupdate_scatter(accl, [d16], v)
                return carry

            lax.fori_loop(0, _CH // 16, stept, 0)

        pltpu.sync_copy(accl, out_hbm.at[wid])

    return pl.kernel(
        body, out_type=out_type, mesh=mesh, scratch_types=scratch,
        compiler_params=pltpu.CompilerParams(use_tc_tiling_on_sc=False,
                                             needs_layout_passes=False))


def _mm_body(x_ref, w_ref, o_ref):
    o_ref[...] = jnp.dot(x_ref[...], w_ref[...],
                         preferred_element_type=jnp.float32)


def _matmul(x, w):
    n, k = x.shape
    m = w.shape[1]
    return pl.pallas_call(
        _mm_body,
        grid=(n // _BM,),
        in_specs=[pl.BlockSpec((_BM, k), lambda i: (i, 0)),
                  pl.BlockSpec((k, m), lambda i: (0, 0))],
        out_specs=pl.BlockSpec((_BM, m), lambda i: (i, 0)),
        out_shape=jax.ShapeDtypeStruct((n, m), jnp.float32),
    )(x, w)


def _ln_gelu(pre, g_ref, be_ref):
    mu = jnp.mean(pre, axis=-1, keepdims=True)
    var = jnp.mean((pre - mu) ** 2, axis=-1, keepdims=True)
    h = (pre - mu) * lax.rsqrt(var + 1e-5) * g_ref[...] + be_ref[...]
    return h * 0.5 * (1.0 + lax.erf(h * _INV_SQRT2))


def _layer1_body(s_ref, dp_ref, x_ref, wr_ref, b_ref, g_ref, be_ref, wn_ref,
                 ho_ref, po_ref, dego_ref):
    st = s_ref[0] + s_ref[1]
    degc = jnp.maximum(jnp.sum(dp_ref[...], axis=0), 1.0)[:, None]
    pre = st / degc + b_ref[...] + jnp.dot(
        x_ref[...], wr_ref[...], preferred_element_type=jnp.float32)
    h = _ln_gelu(pre, g_ref, be_ref)
    ho_ref[...] = h
    po_ref[...] = jnp.dot(h, wn_ref[...], preferred_element_type=jnp.float32)
    dego_ref[...] = degc


def _layer1(s, deg_p, x, wr, b, g, be, wn):
    return pl.pallas_call(
        _layer1_body,
        grid=(_NPAD // _BM,),
        in_specs=[
            pl.BlockSpec((_NC, _BM, 64), lambda i: (0, i, 0)),
            pl.BlockSpec((_NW, _BM), lambda i: (0, i)),
            pl.BlockSpec((_BM, 128), lambda i: (i, 0)),
            pl.BlockSpec((128, 64), lambda i: (0, 0)),
            pl.BlockSpec((1, 64), lambda i: (0, 0)),
            pl.BlockSpec((1, 64), lambda i: (0, 0)),
            pl.BlockSpec((1, 64), lambda i: (0, 0)),
            pl.BlockSpec((64, 32), lambda i: (0, 0)),
        ],
        out_specs=[pl.BlockSpec((_BM, 64), lambda i: (i, 0)),
                   pl.BlockSpec((_BM, 32), lambda i: (i, 0)),
                   pl.BlockSpec((_BM, 1), lambda i: (i, 0))],
        out_shape=[jax.ShapeDtypeStruct((_NPAD, 64), jnp.float32),
                   jax.ShapeDtypeStruct((_NPAD, 32), jnp.float32),
                   jax.ShapeDtypeStruct((_NPAD, 1), jnp.float32)],
    )(s, deg_p, x, wr, b.reshape(1, 64), g.reshape(1, 64), be.reshape(1, 64),
      wn)


def _layer2_body(s_ref, deg_ref, h_ref, wr_ref, b_ref, g_ref, be_ref, wn_ref,
                 ho_ref, po_ref):
    agg = (s_ref[0] + s_ref[1]) / deg_ref[...]
    pre = agg + b_ref[...] + jnp.dot(h_ref[...], wr_ref[...],
                                     preferred_element_type=jnp.float32)
    h = _ln_gelu(pre, g_ref, be_ref)
    ho_ref[...] = h
    po_ref[...] = jnp.dot(h, wn_ref[...],
                          preferred_element_type=jnp.float32)[:, 0]


def _layer2(s, degc, h_prev, wr, b, g, be, wn):
    return pl.pallas_call(
        _layer2_body,
        grid=(_NPAD // _BM,),
        in_specs=[
            pl.BlockSpec((_NC, _BM, 32), lambda i: (0, i, 0)),
            pl.BlockSpec((_BM, 1), lambda i: (i, 0)),
            pl.BlockSpec((_BM, 64), lambda i: (i, 0)),
            pl.BlockSpec((64, 32), lambda i: (0, 0)),
            pl.BlockSpec((1, 32), lambda i: (0, 0)),
            pl.BlockSpec((1, 32), lambda i: (0, 0)),
            pl.BlockSpec((1, 32), lambda i: (0, 0)),
            pl.BlockSpec((32, 1), lambda i: (0, 0)),
        ],
        out_specs=[pl.BlockSpec((_BM, 32), lambda i: (i, 0)),
                   pl.BlockSpec((_BM,), lambda i: (i,))],
        out_shape=[jax.ShapeDtypeStruct((_NPAD, 32), jnp.float32),
                   jax.ShapeDtypeStruct((_NPAD,), jnp.float32)],
    )(s, degc, h_prev, wr, b.reshape(1, 32), g.reshape(1, 32),
      be.reshape(1, 32), wn)


def _final_body(s_ref, deg_ref, h_ref, wr_ref, b_ref, o_ref):
    agg = jnp.sum(s_ref[...], axis=0) / deg_ref[...][:, 0]
    o_ref[...] = agg + b_ref[0, 0] + jnp.dot(
        h_ref[...], wr_ref[...], preferred_element_type=jnp.float32)[:, 0]


def _final(s, degc, h_prev, wr, b):
    return pl.pallas_call(
        _final_body,
        grid=(_NPAD // _BM,),
        in_specs=[
            pl.BlockSpec((_NW, _BM), lambda i: (0, i)),
            pl.BlockSpec((_BM, 1), lambda i: (i, 0)),
            pl.BlockSpec((_BM, 32), lambda i: (i, 0)),
            pl.BlockSpec((32, 1), lambda i: (0, 0)),
            pl.BlockSpec((1, 1), lambda i: (0, 0)),
        ],
        out_specs=pl.BlockSpec((_BM,), lambda i: (i,)),
        out_shape=jax.ShapeDtypeStruct((_NPAD,), jnp.float32),
    )(s, degc, h_prev, wr, b.reshape(1, 1))


def kernel(x, edge_index, W1l, b1, W1r, g1, be1, W2l, b2, W2r, g2, be2,
           W3l, b3, W3r):
    src2 = edge_index[0].reshape(_NCHUNK, _CH)
    dst2 = edge_index[1].reshape(_NCHUNK, _CH)
    src1 = edge_index[0]
    dst1 = edge_index[1]
    xp = jnp.pad(x, ((0, _NPAD - _N), (0, 0)))
    z64 = jnp.zeros((_NPAD, 64), jnp.float32)
    z32 = jnp.zeros((_NPAD, 32), jnp.float32)
    zd = jnp.zeros((_NPAD,), jnp.float32)

    deg_p, = _seg_deg()(dst1, zd)
    p1 = _matmul(xp, W1l)
    s1, = _seg_sum(64)(p1, src2, dst2, z64)
    h1, p2, degc = _layer1(s1, deg_p, xp, W1r, b1, g1, be1, W2l)
    s2, = _seg_sum(32)(p2, src2, dst2, z32)
    h2, p3 = _layer2(s2, degc, h1, W2r, b2, g2, be2, W3l)
    s3, = _seg_scalar()(p3, src1, dst1, zd)
    out = _final(s3, degc, h2, W3r, b3)
    return out[:_N]


# speedup vs baseline: 19.3135x; 1.0989x over previous
import functools

import jax
import jax.numpy as jnp
from jax import lax
from jax.experimental import pallas as pl
from jax.experimental.pallas import tpu as pltpu
from jax.experimental.pallas import tpu_sc as plsc

_N = 10000
_NPAD = 10240
_E = 320000
_NC = 2
_NS = 16
_NW = _NC * _NS
_CH = 128
_NCHUNK = _E // _CH
_CPW = 78
_CHUNK_REM = _NCHUNK - _CPW * _NW
_EPW = _CPW * _CH
_RPT = _NPAD // _NS
_BM = 1024
_INV_SQRT2 = 0.7071067811865476

_G = 2
_NGRP = _CPW // _G
_GCH = _G * _CH


def _seg_sum(D):
    mesh = plsc.VectorSubcoreMesh(core_axis_name="c", subcore_axis_name="s",
                                  num_cores=_NC, num_subcores=_NS)
    out_type = [jax.ShapeDtypeStruct((_NC, _NPAD, D), jnp.float32)]
    scratch = [
        pltpu.VMEM((_CPW + 1, _CH), jnp.int32),
        pltpu.VMEM((_CPW + 1, _CH), jnp.int32),
        [pltpu.VMEM((_GCH, D), jnp.float32)] * 4,
        pltpu.VMEM_SHARED((_NPAD, D), jnp.float32),
        [pltpu.SemaphoreType.DMA] * 4,
        [pltpu.SemaphoreType.DMA] * 4,
    ]

    def body(p_hbm, src_hbm, dst_hbm, z_hbm, out_hbm,
             src_blk, dst_blk, bufs, acc, gsem, ssem):
        c = lax.axis_index("c")
        s = lax.axis_index("s")
        wid = s * _NC + c
        r0 = s * _RPT
        pltpu.sync_copy(z_hbm.at[pl.ds(r0, _RPT)], acc.at[pl.ds(r0, _RPT)])

        row0 = _CPW * wid
        pltpu.sync_copy(src_hbm.at[pl.ds(row0, _CPW)],
                        src_blk.at[pl.ds(0, _CPW)])
        pltpu.sync_copy(dst_hbm.at[pl.ds(row0, _CPW)],
                        dst_blk.at[pl.ds(0, _CPW)])

        @pl.when(wid < _CHUNK_REM)
        def _():
            tr = _CPW * _NW + wid
            pltpu.sync_copy(src_hbm.at[pl.ds(tr, 1)],
                            src_blk.at[pl.ds(_CPW, 1)])
            pltpu.sync_copy(dst_hbm.at[pl.ds(tr, 1)],
                            dst_blk.at[pl.ds(_CPW, 1)])

        plsc.subcore_barrier()

        def fire_gathers(g, i):
            for t in range(_G):
                pltpu.async_copy(p_hbm.at[src_blk.at[g * _G + t]],
                                 bufs[i].at[pl.ds(t * _CH, _CH)], gsem[i])

        def drain_gathers(g, i):
            for t in range(_G):
                pltpu.make_async_copy(p_hbm.at[src_blk.at[g * _G + t]],
                                      bufs[i].at[pl.ds(t * _CH, _CH)],
                                      gsem[i]).wait()

        def fire_scatters(g, i):
            for t in range(_G):
                pltpu.async_copy(bufs[i].at[pl.ds(t * _CH, _CH)],
                                 acc.at[dst_blk.at[g * _G + t]], ssem[i],
                                 add=True)

        def drain_scatters(g, i):
            for t in range(_G):
                pltpu.make_async_copy(bufs[i].at[pl.ds(t * _CH, _CH)],
                                      acc.at[dst_blk.at[g * _G + t]],
                                      ssem[i]).wait()

        fire_gathers(0, 0)
        fire_gathers(1, 1)
        for g in range(4):
            i = g
            i2 = (i + 2) % 4
            drain_gathers(g, i)
            fire_scatters(g, i)
            if g >= 2:
                drain_scatters(g - 2, i2)
            fire_gathers(g + 2, i2)

        def step(k, carry):
            for i in range(4):
                g = 4 * k + i
                i2 = (i + 2) % 4
                drain_gathers(g, i)
                fire_scatters(g, i)
                drain_scatters(g - 2, i2)
                fire_gathers(g + 2, i2)
            return carry

        lax.fori_loop(1, _NGRP // 4, step, 0)

        for g in range(4 * (_NGRP // 4), _NGRP):
            i = g % 4
            i2 = (i + 2) % 4
            drain_gathers(g, i)
            fire_scatters(g, i)
            if g + 2 < _NGRP:
                drain_scatters(g - 2, i2)
                fire_gathers(g + 2, i2)
        for g in range(_NGRP - 4, _NGRP):
            drain_scatters(g, g % 4)

        @pl.when(wid < _CHUNK_REM)
        def _():
            pltpu.async_copy(p_hbm.at[src_blk.at[_CPW]],
                             bufs[0].at[pl.ds(0, _CH)], gsem[0])
            pltpu.make_async_copy(p_hbm.at[src_blk.at[_CPW]],
                                  bufs[0].at[pl.ds(0, _CH)], gsem[0]).wait()
            pltpu.sync_copy(bufs[0].at[pl.ds(0, _CH)],
                            acc.at[dst_blk.at[_CPW]], add=True)

        plsc.subcore_barrier()
        pltpu.sync_copy(acc.at[pl.ds(r0, _RPT)], out_hbm.at[c, pl.ds(r0, _RPT)])

    return pl.kernel(
        body, out_type=out_type, mesh=mesh, scratch_types=scratch,
        compiler_params=pltpu.CompilerParams(use_tc_tiling_on_sc=False))


def _seg_deg():
    mesh = plsc.VectorSubcoreMesh(core_axis_name="c", subcore_axis_name="s",
                                  num_cores=_NC, num_subcores=_NS)
    out_type = [jax.ShapeDtypeStruct((_NW, _NPAD), jnp.float32)]
    scratch = [
        pltpu.VMEM((_NPAD,), jnp.float32),
        pltpu.VMEM((_EPW,), jnp.int32),
        pltpu.VMEM((_CH,), jnp.int32),
    ]

    def body(dst_hbm, z_hbm, out_hbm, accl, dstm, dstt):
        c = lax.axis_index("c")
        s = lax.axis_index("s")
        wid = s * _NC + c
        pltpu.sync_copy(z_hbm, accl)
        e0 = wid * _EPW
        pltpu.sync_copy(dst_hbm.at[pl.ds(e0, _EPW)], dstm)

        @pl.when(wid < _CHUNK_REM)
        def _():
            t0 = _EPW * _NW + _CH * wid
            pltpu.sync_copy(dst_hbm.at[pl.ds(t0, _CH)], dstt)

        ones16 = jnp.ones((16,), jnp.float32)

        def step(i, carry):
            d16 = dstm[pl.ds(i * 16, 16)]
            plsc.addupdate_scatter(accl, [d16], ones16)
            return carry

        lax.fori_loop(0, _EPW // 16, step, 0)

        @pl.when(wid < _CHUNK_REM)
        def _():
            def stept(i, carry):
                d16 = dstt[pl.ds(i * 16, 16)]
                plsc.addupdate_scatter(accl, [d16], ones16)
                return carry

            lax.fori_loop(0, _CH // 16, stept, 0)

        pltpu.sync_copy(accl, out_hbm.at[wid])

    return pl.kernel(
        body, out_type=out_type, mesh=mesh, scratch_types=scratch,
        compiler_params=pltpu.CompilerParams(use_tc_tiling_on_sc=False,
                                             needs_layout_passes=False))


def _seg_scalar():
    mesh = plsc.VectorSubcoreMesh(core_axis_name="c", subcore_axis_name="s",
                                  num_cores=_NC, num_subcores=_NS)
    out_type = [jax.ShapeDtypeStruct((_NW, _NPAD), jnp.float32)]
    scratch = [
        pltpu.VMEM((_NPAD,), jnp.float32),
        pltpu.VMEM((_NPAD,), jnp.float32),
        pltpu.VMEM((_EPW,), jnp.int32),
        pltpu.VMEM((_EPW,), jnp.int32),
        pltpu.VMEM((_CH,), jnp.int32),
        pltpu.VMEM((_CH,), jnp.int32),
    ]

    def body(p_hbm, src_hbm, dst_hbm, z_hbm, out_hbm,
             p3_v, accl, srcm, dstm, srct, dstt):
        c = lax.axis_index("c")
        s = lax.axis_index("s")
        wid = s * _NC + c
        pltpu.sync_copy(p_hbm, p3_v)
        pltpu.sync_copy(z_hbm, accl)
        e0 = wid * _EPW
        pltpu.sync_copy(src_hbm.at[pl.ds(e0, _EPW)], srcm)
        pltpu.sync_copy(dst_hbm.at[pl.ds(e0, _EPW)], dstm)

        @pl.when(wid < _CHUNK_REM)
        def _():
            t0 = _EPW * _NW + _CH * wid
            pltpu.sync_copy(src_hbm.at[pl.ds(t0, _CH)], srct)
            pltpu.sync_copy(dst_hbm.at[pl.ds(t0, _CH)], dstt)

        def step(i, carry):
            s16 = srcm[pl.ds(i * 16, 16)]
            d16 = dstm[pl.ds(i * 16, 16)]
            v = plsc.load_gather(p3_v, [s16])
            plsc.addupdate_scatter(accl, [d16], v)
            return carry

        lax.fori_loop(0, _EPW // 16, step, 0)

        @pl.when(wid < _CHUNK_REM)
        def _():
            def stept(i, carry):
                s16 = srct[pl.ds(i * 16, 16)]
                d16 = dstt[pl.ds(i * 16, 16)]
                v = plsc.load_gather(p3_v, [s16])
                plsc.addupdate_scatter(accl, [d16], v)
                return carry

            lax.fori_loop(0, _CH // 16, stept, 0)

        pltpu.sync_copy(accl, out_hbm.at[wid])

    return pl.kernel(
        body, out_type=out_type, mesh=mesh, scratch_types=scratch,
        compiler_params=pltpu.CompilerParams(use_tc_tiling_on_sc=False,
                                             needs_layout_passes=False))


def _mm_body(x_ref, w_ref, o_ref):
    o_ref[...] = jnp.dot(x_ref[...], w_ref[...],
                         preferred_element_type=jnp.float32)


def _matmul(x, w):
    n, k = x.shape
    m = w.shape[1]
    return pl.pallas_call(
        _mm_body,
        grid=(n // _BM,),
        in_specs=[pl.BlockSpec((_BM, k), lambda i: (i, 0)),
                  pl.BlockSpec((k, m), lambda i: (0, 0))],
        out_specs=pl.BlockSpec((_BM, m), lambda i: (i, 0)),
        out_shape=jax.ShapeDtypeStruct((n, m), jnp.float32),
    )(x, w)


def _ln_gelu(pre, g_ref, be_ref):
    mu = jnp.mean(pre, axis=-1, keepdims=True)
    var = jnp.mean((pre - mu) ** 2, axis=-1, keepdims=True)
    h = (pre - mu) * lax.rsqrt(var + 1e-5) * g_ref[...] + be_ref[...]
    return h * 0.5 * (1.0 + lax.erf(h * _INV_SQRT2))


def _layer1_body(s_ref, dp_ref, x_ref, wr_ref, b_ref, g_ref, be_ref, wn_ref,
                 ho_ref, po_ref, dego_ref):
    st = s_ref[0] + s_ref[1]
    degc = jnp.maximum(jnp.sum(dp_ref[...], axis=0), 1.0)[:, None]
    pre = st / degc + b_ref[...] + jnp.dot(
        x_ref[...], wr_ref[...], preferred_element_type=jnp.float32)
    h = _ln_gelu(pre, g_ref, be_ref)
    ho_ref[...] = h
    po_ref[...] = jnp.dot(h, wn_ref[...], preferred_element_type=jnp.float32)
    dego_ref[...] = degc


def _layer1(s, deg_p, x, wr, b, g, be, wn):
    return pl.pallas_call(
        _layer1_body,
        grid=(_NPAD // _BM,),
        in_specs=[
            pl.BlockSpec((_NC, _BM, 64), lambda i: (0, i, 0)),
            pl.BlockSpec((_NW, _BM), lambda i: (0, i)),
            pl.BlockSpec((_BM, 128), lambda i: (i, 0)),
            pl.BlockSpec((128, 64), lambda i: (0, 0)),
            pl.BlockSpec((1, 64), lambda i: (0, 0)),
            pl.BlockSpec((1, 64), lambda i: (0, 0)),
            pl.BlockSpec((1, 64), lambda i: (0, 0)),
            pl.BlockSpec((64, 32), lambda i: (0, 0)),
        ],
        out_specs=[pl.BlockSpec((_BM, 64), lambda i: (i, 0)),
                   pl.BlockSpec((_BM, 32), lambda i: (i, 0)),
                   pl.BlockSpec((_BM, 1), lambda i: (i, 0))],
        out_shape=[jax.ShapeDtypeStruct((_NPAD, 64), jnp.float32),
                   jax.ShapeDtypeStruct((_NPAD, 32), jnp.float32),
                   jax.ShapeDtypeStruct((_NPAD, 1), jnp.float32)],
    )(s, deg_p, x, wr, b.reshape(1, 64), g.reshape(1, 64), be.reshape(1, 64),
      wn)


def _layer2_body(s_ref, deg_ref, h_ref, wr_ref, b_ref, g_ref, be_ref, wn_ref,
                 ho_ref, po_ref):
    agg = (s_ref[0] + s_ref[1]) / deg_ref[...]
    pre = agg + b_ref[...] + jnp.dot(h_ref[...], wr_ref[...],
                                     preferred_element_type=jnp.float32)
    h = _ln_gelu(pre, g_ref, be_ref)
    ho_ref[...] = h
    po_ref[...] = jnp.dot(h, wn_ref[...],
                          preferred_element_type=jnp.float32)[:, 0]


def _layer2(s, degc, h_prev, wr, b, g, be, wn):
    return pl.pallas_call(
        _layer2_body,
        grid=(_NPAD // _BM,),
        in_specs=[
            pl.BlockSpec((_NC, _BM, 32), lambda i: (0, i, 0)),
            pl.BlockSpec((_BM, 1), lambda i: (i, 0)),
            pl.BlockSpec((_BM, 64), lambda i: (i, 0)),
            pl.BlockSpec((64, 32), lambda i: (0, 0)),
            pl.BlockSpec((1, 32), lambda i: (0, 0)),
            pl.BlockSpec((1, 32), lambda i: (0, 0)),
            pl.BlockSpec((1, 32), lambda i: (0, 0)),
            pl.BlockSpec((32, 1), lambda i: (0, 0)),
        ],
        out_specs=[pl.BlockSpec((_BM, 32), lambda i: (i, 0)),
                   pl.BlockSpec((_BM,), lambda i: (i,))],
        out_shape=[jax.ShapeDtypeStruct((_NPAD, 32), jnp.float32),
                   jax.ShapeDtypeStruct((_NPAD,), jnp.float32)],
    )(s, degc, h_prev, wr, b.reshape(1, 32), g.reshape(1, 32),
      be.reshape(1, 32), wn)


def _final_body(s_ref, deg_ref, h_ref, wr_ref, b_ref, o_ref):
    agg = jnp.sum(s_ref[...], axis=0) / deg_ref[...][:, 0]
    o_ref[...] = agg + b_ref[0, 0] + jnp.dot(
        h_ref[...], wr_ref[...], preferred_element_type=jnp.float32)[:, 0]


def _final(s, degc, h_prev, wr, b):
    return pl.pallas_call(
        _final_body,
        grid=(_NPAD // _BM,),
        in_specs=[
            pl.BlockSpec((_NW, _BM), lambda i: (0, i)),
            pl.BlockSpec((_BM, 1), lambda i: (i, 0)),
            pl.BlockSpec((_BM, 32), lambda i: (i, 0)),
            pl.BlockSpec((32, 1), lambda i: (0, 0)),
            pl.BlockSpec((1, 1), lambda i: (0, 0)),
        ],
        out_specs=pl.BlockSpec((_BM,), lambda i: (i,)),
        out_shape=jax.ShapeDtypeStruct((_NPAD,), jnp.float32),
    )(s, degc, h_prev, wr, b.reshape(1, 1))


def kernel(x, edge_index, W1l, b1, W1r, g1, be1, W2l, b2, W2r, g2, be2,
           W3l, b3, W3r):
    src2 = edge_index[0].reshape(_NCHUNK, _CH)
    dst2 = edge_index[1].reshape(_NCHUNK, _CH)
    src1 = edge_index[0]
    dst1 = edge_index[1]
    xp = jnp.pad(x, ((0, _NPAD - _N), (0, 0)))
    z64 = jnp.zeros((_NPAD, 64), jnp.float32)
    z32 = jnp.zeros((_NPAD, 32), jnp.float32)
    zd = jnp.zeros((_NPAD,), jnp.float32)

    deg_p, = _seg_deg()(dst1, zd)
    p1 = _matmul(xp, W1l)
    s1, = _seg_sum(64)(p1, src2, dst2, z64)
    h1, p2, degc = _layer1(s1, deg_p, xp, W1r, b1, g1, be1, W2l)
    s2, = _seg_sum(32)(p2, src2, dst2, z32)
    h2, p3 = _layer2(s2, degc, h1, W2r, b2, g2, be2, W3l)
    s3, = _seg_scalar()(p3, src1, dst1, zd)
    out = _final(s3, degc, h2, W3r, b3)
    return out[:_N]


# speedup vs baseline: 20.1896x; 1.0454x over previous
import functools

import jax
import jax.numpy as jnp
from jax import lax
from jax.experimental import pallas as pl
from jax.experimental.pallas import tpu as pltpu
from jax.experimental.pallas import tpu_sc as plsc

_N = 10000
_NPAD = 10240
_E = 320000
_NC = 2
_NS = 16
_NW = _NC * _NS
_CH = 128
_NCHUNK = _E // _CH
_CPW = 78
_CHUNK_REM = _NCHUNK - _CPW * _NW
_EPW = _CPW * _CH
_RPT = _NPAD // _NS
_BM = 2048
_INV_SQRT2 = 0.7071067811865476

_G = 2
_NGRP = _CPW // _G
_GCH = _G * _CH


def _seg_sum(D):
    mesh = plsc.VectorSubcoreMesh(core_axis_name="c", subcore_axis_name="s",
                                  num_cores=_NC, num_subcores=_NS)
    out_type = [jax.ShapeDtypeStruct((_NC, _NPAD, D), jnp.float32)]
    scratch = [
        pltpu.VMEM((_CPW + 1, _CH), jnp.int32),
        pltpu.VMEM((_CPW + 1, _CH), jnp.int32),
        [pltpu.VMEM((_GCH, D), jnp.float32)] * 4,
        pltpu.VMEM_SHARED((_NPAD, D), jnp.float32),
        [pltpu.SemaphoreType.DMA] * 4,
        [pltpu.SemaphoreType.DMA] * 4,
    ]

    def body(p_hbm, src_hbm, dst_hbm, z_hbm, out_hbm,
             src_blk, dst_blk, bufs, acc, gsem, ssem):
        c = lax.axis_index("c")
        s = lax.axis_index("s")
        wid = s * _NC + c
        r0 = s * _RPT
        pltpu.sync_copy(z_hbm.at[pl.ds(r0, _RPT)], acc.at[pl.ds(r0, _RPT)])

        row0 = _CPW * wid
        pltpu.sync_copy(src_hbm.at[pl.ds(row0, _CPW)],
                        src_blk.at[pl.ds(0, _CPW)])
        pltpu.sync_copy(dst_hbm.at[pl.ds(row0, _CPW)],
                        dst_blk.at[pl.ds(0, _CPW)])

        @pl.when(wid < _CHUNK_REM)
        def _():
            tr = _CPW * _NW + wid
            pltpu.sync_copy(src_hbm.at[pl.ds(tr, 1)],
                            src_blk.at[pl.ds(_CPW, 1)])
            pltpu.sync_copy(dst_hbm.at[pl.ds(tr, 1)],
                            dst_blk.at[pl.ds(_CPW, 1)])

        plsc.subcore_barrier()

        def fire_gathers(g, i):
            for t in range(_G):
                pltpu.async_copy(p_hbm.at[src_blk.at[g * _G + t]],
                                 bufs[i].at[pl.ds(t * _CH, _CH)], gsem[i])

        def drain_gathers(g, i):
            for t in range(_G):
                pltpu.make_async_copy(p_hbm.at[src_blk.at[g * _G + t]],
                                      bufs[i].at[pl.ds(t * _CH, _CH)],
                                      gsem[i]).wait()

        def fire_scatters(g, i):
            for t in range(_G):
                pltpu.async_copy(bufs[i].at[pl.ds(t * _CH, _CH)],
                                 acc.at[dst_blk.at[g * _G + t]], ssem[i],
                                 add=True)

        def drain_scatters(g, i):
            for t in range(_G):
                pltpu.make_async_copy(bufs[i].at[pl.ds(t * _CH, _CH)],
                                      acc.at[dst_blk.at[g * _G + t]],
                                      ssem[i]).wait()

        fire_gathers(0, 0)
        fire_gathers(1, 1)
        for g in range(4):
            i = g
            i2 = (i + 2) % 4
            drain_gathers(g, i)
            fire_scatters(g, i)
            if g >= 2:
                drain_scatters(g - 2, i2)
            fire_gathers(g + 2, i2)

        def step(k, carry):
            for i in range(4):
                g = 4 * k + i
                i2 = (i + 2) % 4
                drain_gathers(g, i)
                fire_scatters(g, i)
                drain_scatters(g - 2, i2)
                fire_gathers(g + 2, i2)
            return carry

        lax.fori_loop(1, _NGRP // 4, step, 0)

        for g in range(4 * (_NGRP // 4), _NGRP):
            i = g % 4
            i2 = (i + 2) % 4
            drain_gathers(g, i)
            fire_scatters(g, i)
            if g + 2 < _NGRP:
                drain_scatters(g - 2, i2)
                fire_gathers(g + 2, i2)
        for g in range(_NGRP - 4, _NGRP):
            drain_scatters(g, g % 4)

        @pl.when(wid < _CHUNK_REM)
        def _():
            pltpu.async_copy(p_hbm.at[src_blk.at[_CPW]],
                             bufs[0].at[pl.ds(0, _CH)], gsem[0])
            pltpu.make_async_copy(p_hbm.at[src_blk.at[_CPW]],
                                  bufs[0].at[pl.ds(0, _CH)], gsem[0]).wait()
            pltpu.sync_copy(bufs[0].at[pl.ds(0, _CH)],
                            acc.at[dst_blk.at[_CPW]], add=True)

        plsc.subcore_barrier()
        pltpu.sync_copy(acc.at[pl.ds(r0, _RPT)], out_hbm.at[c, pl.ds(r0, _RPT)])

    return pl.kernel(
        body, out_type=out_type, mesh=mesh, scratch_types=scratch,
        compiler_params=pltpu.CompilerParams(use_tc_tiling_on_sc=False))


def _seg_deg():
    mesh = plsc.VectorSubcoreMesh(core_axis_name="c", subcore_axis_name="s",
                                  num_cores=_NC, num_subcores=_NS)
    out_type = [jax.ShapeDtypeStruct((_NW, _NPAD), jnp.float32)]
    scratch = [
        pltpu.VMEM((_NPAD,), jnp.float32),
        pltpu.VMEM((_EPW,), jnp.int32),
        pltpu.VMEM((_CH,), jnp.int32),
    ]

    def body(dst_hbm, z_hbm, out_hbm, accl, dstm, dstt):
        c = lax.axis_index("c")
        s = lax.axis_index("s")
        wid = s * _NC + c
        pltpu.sync_copy(z_hbm, accl)
        e0 = wid * _EPW
        pltpu.sync_copy(dst_hbm.at[pl.ds(e0, _EPW)], dstm)

        @pl.when(wid < _CHUNK_REM)
        def _():
            t0 = _EPW * _NW + _CH * wid
            pltpu.sync_copy(dst_hbm.at[pl.ds(t0, _CH)], dstt)

        ones16 = jnp.ones((16,), jnp.float32)

        def step(i, carry):
            d16 = dstm[pl.ds(i * 16, 16)]
            plsc.addupdate_scatter(accl, [d16], ones16)
            return carry

        lax.fori_loop(0, _EPW // 16, step, 0)

        @pl.when(wid < _CHUNK_REM)
        def _():
            def stept(i, carry):
                d16 = dstt[pl.ds(i * 16, 16)]
                plsc.addupdate_scatter(accl, [d16], ones16)
                return carry

            lax.fori_loop(0, _CH // 16, stept, 0)

        pltpu.sync_copy(accl, out_hbm.at[wid])

    return pl.kernel(
        body, out_type=out_type, mesh=mesh, scratch_types=scratch,
        compiler_params=pltpu.CompilerParams(use_tc_tiling_on_sc=False,
                                             needs_layout_passes=False))


def _seg_scalar():
    mesh = plsc.VectorSubcoreMesh(core_axis_name="c", subcore_axis_name="s",
                                  num_cores=_NC, num_subcores=_NS)
    out_type = [jax.ShapeDtypeStruct((_NW, _NPAD), jnp.float32)]
    scratch = [
        pltpu.VMEM((_NPAD,), jnp.float32),
        pltpu.VMEM((_NPAD,), jnp.float32),
        pltpu.VMEM((_EPW,), jnp.int32),
        pltpu.VMEM((_EPW,), jnp.int32),
        pltpu.VMEM((_CH,), jnp.int32),
        pltpu.VMEM((_CH,), jnp.int32),
    ]

    def body(p_hbm, src_hbm, dst_hbm, z_hbm, out_hbm,
             p3_v, accl, srcm, dstm, srct, dstt):
        c = lax.axis_index("c")
        s = lax.axis_index("s")
        wid = s * _NC + c
        pltpu.sync_copy(p_hbm, p3_v)
        pltpu.sync_copy(z_hbm, accl)
        e0 = wid * _EPW
        pltpu.sync_copy(src_hbm.at[pl.ds(e0, _EPW)], srcm)
        pltpu.sync_copy(dst_hbm.at[pl.ds(e0, _EPW)], dstm)

        @pl.when(wid < _CHUNK_REM)
        def _():
            t0 = _EPW * _NW + _CH * wid
            pltpu.sync_copy(src_hbm.at[pl.ds(t0, _CH)], srct)
            pltpu.sync_copy(dst_hbm.at[pl.ds(t0, _CH)], dstt)

        def step(i, carry):
            s16 = srcm[pl.ds(i * 16, 16)]
            d16 = dstm[pl.ds(i * 16, 16)]
            v = plsc.load_gather(p3_v, [s16])
            plsc.addupdate_scatter(accl, [d16], v)
            return carry

        lax.fori_loop(0, _EPW // 16, step, 0)

        @pl.when(wid < _CHUNK_REM)
        def _():
            def stept(i, carry):
                s16 = srct[pl.ds(i * 16, 16)]
                d16 = dstt[pl.ds(i * 16, 16)]
                v = plsc.load_gather(p3_v, [s16])
                plsc.addupdate_scatter(accl, [d16], v)
                return carry

            lax.fori_loop(0, _CH // 16, stept, 0)

        pltpu.sync_copy(accl, out_hbm.at[wid])

    return pl.kernel(
        body, out_type=out_type, mesh=mesh, scratch_types=scratch,
        compiler_params=pltpu.CompilerParams(use_tc_tiling_on_sc=False,
                                             needs_layout_passes=False))


def _mm_body(x_ref, w_ref, o_ref):
    o_ref[...] = jnp.dot(x_ref[...], w_ref[...],
                         preferred_element_type=jnp.float32)


def _matmul(x, w):
    n, k = x.shape
    m = w.shape[1]
    return pl.pallas_call(
        _mm_body,
        grid=(n // _BM,),
        in_specs=[pl.BlockSpec((_BM, k), lambda i: (i, 0)),
                  pl.BlockSpec((k, m), lambda i: (0, 0))],
        out_specs=pl.BlockSpec((_BM, m), lambda i: (i, 0)),
        out_shape=jax.ShapeDtypeStruct((n, m), jnp.float32),
    )(x, w)


def _ln_gelu(pre, g_ref, be_ref):
    mu = jnp.mean(pre, axis=-1, keepdims=True)
    var = jnp.mean((pre - mu) ** 2, axis=-1, keepdims=True)
    h = (pre - mu) * lax.rsqrt(var + 1e-5) * g_ref[...] + be_ref[...]
    return h * 0.5 * (1.0 + lax.erf(h * _INV_SQRT2))


def _layer1_body(s_ref, dp_ref, x_ref, wr_ref, b_ref, g_ref, be_ref, wn_ref,
                 ho_ref, po_ref, dego_ref):
    st = s_ref[0] + s_ref[1]
    degc = jnp.maximum(jnp.sum(dp_ref[...], axis=0), 1.0)[:, None]
    pre = st / degc + b_ref[...] + jnp.dot(
        x_ref[...], wr_ref[...], preferred_element_type=jnp.float32)
    h = _ln_gelu(pre, g_ref, be_ref)
    ho_ref[...] = h
    po_ref[...] = jnp.dot(h, wn_ref[...], preferred_element_type=jnp.float32)
    dego_ref[...] = degc


def _layer1(s, deg_p, x, wr, b, g, be, wn):
    return pl.pallas_call(
        _layer1_body,
        grid=(_NPAD // _BM,),
        in_specs=[
            pl.BlockSpec((_NC, _BM, 64), lambda i: (0, i, 0)),
            pl.BlockSpec((_NW, _BM), lambda i: (0, i)),
            pl.BlockSpec((_BM, 128), lambda i: (i, 0)),
            pl.BlockSpec((128, 64), lambda i: (0, 0)),
            pl.BlockSpec((1, 64), lambda i: (0, 0)),
            pl.BlockSpec((1, 64), lambda i: (0, 0)),
            pl.BlockSpec((1, 64), lambda i: (0, 0)),
            pl.BlockSpec((64, 32), lambda i: (0, 0)),
        ],
        out_specs=[pl.BlockSpec((_BM, 64), lambda i: (i, 0)),
                   pl.BlockSpec((_BM, 32), lambda i: (i, 0)),
                   pl.BlockSpec((_BM, 1), lambda i: (i, 0))],
        out_shape=[jax.ShapeDtypeStruct((_NPAD, 64), jnp.float32),
                   jax.ShapeDtypeStruct((_NPAD, 32), jnp.float32),
                   jax.ShapeDtypeStruct((_NPAD, 1), jnp.float32)],
    )(s, deg_p, x, wr, b.reshape(1, 64), g.reshape(1, 64), be.reshape(1, 64),
      wn)


def _layer2_body(s_ref, deg_ref, h_ref, wr_ref, b_ref, g_ref, be_ref, wn_ref,
                 ho_ref, po_ref):
    agg = (s_ref[0] + s_ref[1]) / deg_ref[...]
    pre = agg + b_ref[...] + jnp.dot(h_ref[...], wr_ref[...],
                                     preferred_element_type=jnp.float32)
    h = _ln_gelu(pre, g_ref, be_ref)
    ho_ref[...] = h
    po_ref[...] = jnp.dot(h, wn_ref[...],
                          preferred_element_type=jnp.float32)[:, 0]


def _layer2(s, degc, h_prev, wr, b, g, be, wn):
    return pl.pallas_call(
        _layer2_body,
        grid=(_NPAD // _BM,),
        in_specs=[
            pl.BlockSpec((_NC, _BM, 32), lambda i: (0, i, 0)),
            pl.BlockSpec((_BM, 1), lambda i: (i, 0)),
            pl.BlockSpec((_BM, 64), lambda i: (i, 0)),
            pl.BlockSpec((64, 32), lambda i: (0, 0)),
            pl.BlockSpec((1, 32), lambda i: (0, 0)),
            pl.BlockSpec((1, 32), lambda i: (0, 0)),
            pl.BlockSpec((1, 32), lambda i: (0, 0)),
            pl.BlockSpec((32, 1), lambda i: (0, 0)),
        ],
        out_specs=[pl.BlockSpec((_BM, 32), lambda i: (i, 0)),
                   pl.BlockSpec((_BM,), lambda i: (i,))],
        out_shape=[jax.ShapeDtypeStruct((_NPAD, 32), jnp.float32),
                   jax.ShapeDtypeStruct((_NPAD,), jnp.float32)],
    )(s, degc, h_prev, wr, b.reshape(1, 32), g.reshape(1, 32),
      be.reshape(1, 32), wn)


def _final_body(s_ref, deg_ref, h_ref, wr_ref, b_ref, o_ref):
    agg = jnp.sum(s_ref[...], axis=0) / deg_ref[...][:, 0]
    o_ref[...] = agg + b_ref[0, 0] + jnp.dot(
        h_ref[...], wr_ref[...], preferred_element_type=jnp.float32)[:, 0]


def _final(s, degc, h_prev, wr, b):
    return pl.pallas_call(
        _final_body,
        grid=(_NPAD // _BM,),
        in_specs=[
            pl.BlockSpec((_NW, _BM), lambda i: (0, i)),
            pl.BlockSpec((_BM, 1), lambda i: (i, 0)),
            pl.BlockSpec((_BM, 32), lambda i: (i, 0)),
            pl.BlockSpec((32, 1), lambda i: (0, 0)),
            pl.BlockSpec((1, 1), lambda i: (0, 0)),
        ],
        out_specs=pl.BlockSpec((_BM,), lambda i: (i,)),
        out_shape=jax.ShapeDtypeStruct((_NPAD,), jnp.float32),
    )(s, degc, h_prev, wr, b.reshape(1, 1))


def kernel(x, edge_index, W1l, b1, W1r, g1, be1, W2l, b2, W2r, g2, be2,
           W3l, b3, W3r):
    src2 = edge_index[0].reshape(_NCHUNK, _CH)
    dst2 = edge_index[1].reshape(_NCHUNK, _CH)
    src1 = edge_index[0]
    dst1 = edge_index[1]
    xp = jnp.pad(x, ((0, _NPAD - _N), (0, 0)))
    z64 = jnp.zeros((_NPAD, 64), jnp.float32)
    z32 = jnp.zeros((_NPAD, 32), jnp.float32)
    zd = jnp.zeros((_NPAD,), jnp.float32)

    deg_p, = _seg_deg()(dst1, zd)
    p1 = _matmul(xp, W1l)
    s1, = _seg_sum(64)(p1, src2, dst2, z64)
    h1, p2, degc = _layer1(s1, deg_p, xp, W1r, b1, g1, be1, W2l)
    s2, = _seg_sum(32)(p2, src2, dst2, z32)
    h2, p3 = _layer2(s2, degc, h1, W2r, b2, g2, be2, W3l)
    s3, = _seg_scalar()(p3, src1, dst1, zd)
    out = _final(s3, degc, h2, W3r, b3)
    return out[:_N]
